# baseline (speedup 1.0000x reference)
import functools

import jax
import jax.numpy as jnp
from jax import lax
from jax.experimental import pallas as pl
from jax.experimental.pallas import tpu as pltpu
from jax.experimental.pallas import tpu_sc as plsc

N = 10000
E = 160000
D = 128

NC = 2
NS = 16
CHUNK = 128
CH = 80
EPW = CH * CHUNK
N_PAD = 10240
RPT = N_PAD // NS


HCH = CH // 4

_MESH = plsc.VectorSubcoreMesh(core_axis_name="c", subcore_axis_name="s")


def _make_agg(with_deg):
  if with_deg:
    out_type = [jax.ShapeDtypeStruct((NC, N_PAD, D), jnp.float32),
                jax.ShapeDtypeStruct((NC, N_PAD, D), jnp.float32)]
  else:
    out_type = jax.ShapeDtypeStruct((NC, N_PAD, D), jnp.float32)
  scratch = [
      pltpu.VMEM_SHARED((N_PAD, D), jnp.float32),
      pltpu.VMEM((HCH, CHUNK), jnp.int32),
      pltpu.VMEM((HCH, CHUNK), jnp.int32),
  ] + [pltpu.VMEM((CHUNK, D), jnp.float32)] * 2 + [
      pltpu.SemaphoreType.DMA] * 4

  def body(*refs):
    if with_deg:
      (x_hbm, src_hbm, dst_hbm, const_hbm, s_out, deg_out,
       acc, src_v, dst_v, r0b, r1b, g0, g1, s0b, s1b) = refs
    else:
      (x_hbm, src_hbm, dst_hbm, s_out,
       acc, src_v, dst_v, r0b, r1b, g0, g1, s0b, s1b) = refs
    rows = [r0b, r1b]
    gsem = [g0, g1]
    ssem = [s0b, s1b]
    rows0 = r0b
    c = lax.axis_index("c")
    s = lax.axis_index("s")
    r0 = s * RPT
    zv = jnp.zeros((16,), jnp.float32)

    def zrow(i, carry):
      for k in range(D // 16):
        rows0[i, pl.ds(k * 16, 16)] = zv
      return carry
    def zero_acc():
      lax.fori_loop(0, CHUNK, zrow, 0)
      def zacc(t, carry):
        pltpu.sync_copy(rows0, acc.at[pl.ds(r0 + t * CHUNK, CHUNK)])
        return carry
      lax.fori_loop(0, RPT // CHUNK, zacc, 0)
    zero_acc()
    plsc.subcore_barrier()

    def pair(p, carry):
      for b in range(2):
        j = 2 * p + b
        pltpu.make_async_copy(x_hbm.at[src_v.at[j]], rows[b], gsem[b]).wait()
        pltpu.async_copy(rows[b], acc.at[dst_v.at[j]], ssem[b], add=True)
      for b in range(2):
        j = 2 * p + b

        @pl.when(j + 2 < HCH)
        def _():
          pltpu.make_async_copy(rows[b], acc.at[dst_v.at[j]], ssem[b]).wait()
          pltpu.async_copy(x_hbm.at[src_v.at[j + 2]], rows[b], gsem[b])
      return carry

    for hh in range(4):
      pltpu.sync_copy(src_hbm.at[c, s, hh], src_v)
      pltpu.sync_copy(dst_hbm.at[c, s, hh], dst_v)
      for b in range(2):
        pltpu.async_copy(x_hbm.at[src_v.at[b]], rows[b], gsem[b])
      lax.fori_loop(0, HCH // 2, pair, 0)
      for b in range(2):
        pltpu.make_async_copy(
            rows[b], acc.at[dst_v.at[HCH - 2 + b]], ssem[b]).wait()
    plsc.subcore_barrier()

    pltpu.sync_copy(acc.at[pl.ds(r0, RPT)], s_out.at[c, pl.ds(r0, RPT)])

    if with_deg:
      pltpu.sync_copy(const_hbm, rows[1])
      zero_acc()
      plsc.subcore_barrier()

      def dpair(p, carry):
        for b in range(2):
          j = 2 * p + b

          @pl.when(p > 0)
          def _():
            pltpu.make_async_copy(rows[1], acc.at[dst_v.at[j]],
                                  ssem[b]).wait()
          pltpu.async_copy(rows[1], acc.at[dst_v.at[j]], ssem[b], add=True)
        return carry

      for hh in range(4):
        pltpu.sync_copy(dst_hbm.at[c, s, hh], dst_v)
        lax.fori_loop(0, HCH // 2, dpair, 0)
        for b in range(2):
          pltpu.make_async_copy(rows[1], acc.at[dst_v.at[0]], ssem[b]).wait()
      plsc.subcore_barrier()
      pltpu.sync_copy(acc.at[pl.ds(r0, RPT)], deg_out.at[c, pl.ds(r0, RPT)])

  return pl.kernel(body, mesh=_MESH, out_type=out_type, scratch_types=scratch)


_agg_plain = _make_agg(False)
_agg_deg = _make_agg(True)


def _layer_body(relu, s0_ref, s1_ref, d0_ref, d1_ref, w_ref, b_ref, o_ref):
  d0 = d0_ref[:, 0:1]
  d1 = d1_ref[:, 0:1]
  a0 = s0_ref[...] / jnp.maximum(d0, 1.0)
  a1 = s1_ref[...] / jnp.maximum(d1, 1.0)
  h = jnp.dot(a0, w_ref[0], preferred_element_type=jnp.float32)
  h = h + jnp.dot(a1, w_ref[1], preferred_element_type=jnp.float32)
  h = h + b_ref[0, 0:1, :] * (d0 > 0) + b_ref[0, 1:2, :] * (d1 > 0)
  if relu:
    h = jnp.where(h >= 0, h, 0.01 * h)
  o_ref[...] = h


def _layer_tc(s0, s1, d0, d1, w_pack, b_pack, relu):
  blk = 1024
  grid = (N_PAD // blk,)
  return pl.pallas_call(
      functools.partial(_layer_body, relu),
      grid=grid,
      in_specs=[
          pl.BlockSpec((blk, D), lambda i: (i, 0)),
          pl.BlockSpec((blk, D), lambda i: (i, 0)),
          pl.BlockSpec((blk, D), lambda i: (i, 0)),
          pl.BlockSpec((blk, D), lambda i: (i, 0)),
          pl.BlockSpec((2, D, D), lambda i: (0, 0, 0)),
          pl.BlockSpec((1, 8, D), lambda i: (0, 0, 0)),
      ],
      out_specs=pl.BlockSpec((blk, D), lambda i: (i, 0)),
      out_shape=jax.ShapeDtypeStruct((N_PAD, D), jnp.float32),
  )(s0, s1, d0, d1, w_pack, b_pack)


def _pack_idx(ei):
  src = ei[0].astype(jnp.int32)
  dst = ei[1].astype(jnp.int32)
  pad = NS * EPW - E
  src_p = jnp.concatenate([src, jnp.zeros((pad,), jnp.int32)])
  dst_p = jnp.concatenate([dst, jnp.full((pad,), N, jnp.int32)])
  return (src_p.reshape(NS, CH // HCH, HCH, CHUNK),
          dst_p.reshape(NS, CH // HCH, HCH, CHUNK))


def kernel(embed, edge_index_rel0, edge_index_rel1,
           W1_rel0, b1_rel0, W1_rel1, b1_rel1,
           W2_rel0, b2_rel0, W2_rel1, b2_rel1):
  s0s, d0s = _pack_idx(edge_index_rel0)
  s1s, d1s = _pack_idx(edge_index_rel1)
  src_all = jnp.stack([s0s, s1s])
  dst_all = jnp.stack([d0s, d1s])

  const128 = jnp.zeros((CHUNK, D), jnp.float32).at[:, 0].set(1.0)
  sums1, deg = _agg_deg(embed, src_all, dst_all, const128)

  w1 = jnp.stack([W1_rel0, W1_rel1])
  b1 = jnp.stack([b1_rel0, b1_rel1]).reshape(2, D)
  b1 = jnp.concatenate([b1, jnp.zeros((6, D), jnp.float32)]).reshape(1, 8, D)
  h = _layer_tc(sums1[0], sums1[1], deg[0], deg[1], w1, b1, True)

  sums2 = _agg_plain(h, src_all, dst_all)

  w2 = jnp.stack([W2_rel0, W2_rel1])
  b2 = jnp.stack([b2_rel0, b2_rel1]).reshape(2, D)
  b2 = jnp.concatenate([b2, jnp.zeros((6, D), jnp.float32)]).reshape(1, 8, D)
  out = _layer_tc(sums2[0], sums2[1], deg[0], deg[1], w2, b2, False)
  return out[:N]


# speedup vs baseline: 1.0449x; 1.0449x over previous
import functools

import jax
import jax.numpy as jnp
from jax import lax
from jax.experimental import pallas as pl
from jax.experimental.pallas import tpu as pltpu
from jax.experimental.pallas import tpu_sc as plsc

N = 10000
E = 160000
D = 128

NC = 2
NS = 16
CHUNK = 128
CH = 80
EPW = CH * CHUNK
N_PAD = 10240
RPT = N_PAD // NS


HCH = CH // 4

_MESH = plsc.VectorSubcoreMesh(core_axis_name="c", subcore_axis_name="s")


def _make_agg(with_deg):
  if with_deg:
    out_type = [jax.ShapeDtypeStruct((NC, N_PAD, D), jnp.float32),
                jax.ShapeDtypeStruct((NC, N_PAD, D), jnp.float32)]
  else:
    out_type = jax.ShapeDtypeStruct((NC, N_PAD, D), jnp.float32)
  scratch = [
      pltpu.VMEM_SHARED((N_PAD, D), jnp.float32),
      pltpu.VMEM((HCH, CHUNK), jnp.int32),
      pltpu.VMEM((HCH, CHUNK), jnp.int32),
  ] + [pltpu.VMEM((CHUNK, D), jnp.float32)] * 2 + [
      pltpu.SemaphoreType.DMA] * 4

  def body(*refs):
    if with_deg:
      (x_hbm, src_hbm, dst_hbm, const_hbm, s_out, deg_out,
       acc, src_v, dst_v, r0b, r1b, g0, g1, s0b, s1b) = refs
    else:
      (x_hbm, src_hbm, dst_hbm, s_out,
       acc, src_v, dst_v, r0b, r1b, g0, g1, s0b, s1b) = refs
    rows = [r0b, r1b]
    gsem = [g0, g1]
    ssem = [s0b, s1b]
    rows0 = r0b
    c = lax.axis_index("c")
    s = lax.axis_index("s")
    r0 = s * RPT
    zv = jnp.zeros((16,), jnp.float32)

    def zrow(i, carry):
      for k in range(D // 16):
        rows0[i, pl.ds(k * 16, 16)] = zv
      return carry
    def zero_acc():
      lax.fori_loop(0, CHUNK, zrow, 0)
      def zacc(t, carry):
        pltpu.sync_copy(rows0, acc.at[pl.ds(r0 + t * CHUNK, CHUNK)])
        return carry
      lax.fori_loop(0, RPT // CHUNK, zacc, 0)
    zero_acc()
    plsc.subcore_barrier()

    def pair(p, carry):
      j0 = 2 * p
      pltpu.async_copy(x_hbm.at[src_v.at[j0 + 1]], rows[1], gsem[1])
      pltpu.make_async_copy(x_hbm.at[src_v.at[j0]], rows[0], gsem[0]).wait()
      pltpu.sync_copy(rows[0], acc.at[dst_v.at[j0]], add=True)

      @pl.when(j0 + 2 < HCH)
      def _():
        pltpu.async_copy(x_hbm.at[src_v.at[j0 + 2]], rows[0], gsem[0])

      pltpu.make_async_copy(x_hbm.at[src_v.at[j0 + 1]], rows[1],
                            gsem[1]).wait()
      pltpu.sync_copy(rows[1], acc.at[dst_v.at[j0 + 1]], add=True)
      return carry

    for hh in range(4):
      pltpu.sync_copy(src_hbm.at[c, s, hh], src_v)
      pltpu.sync_copy(dst_hbm.at[c, s, hh], dst_v)
      pltpu.async_copy(x_hbm.at[src_v.at[0]], rows[0], gsem[0])
      lax.fori_loop(0, HCH // 2, pair, 0)
    plsc.subcore_barrier()

    pltpu.sync_copy(acc.at[pl.ds(r0, RPT)], s_out.at[c, pl.ds(r0, RPT)])

    if with_deg:
      pltpu.sync_copy(const_hbm, rows[1])
      zero_acc()
      plsc.subcore_barrier()

      def dpair(p, carry):
        for b in range(2):
          j = 2 * p + b

          @pl.when(p > 0)
          def _():
            pltpu.make_async_copy(rows[1], acc.at[dst_v.at[j]],
                                  ssem[b]).wait()
          pltpu.async_copy(rows[1], acc.at[dst_v.at[j]], ssem[b], add=True)
        return carry

      for hh in range(4):
        pltpu.sync_copy(dst_hbm.at[c, s, hh], dst_v)
        lax.fori_loop(0, HCH // 2, dpair, 0)
        for b in range(2):
          pltpu.make_async_copy(rows[1], acc.at[dst_v.at[0]], ssem[b]).wait()
      plsc.subcore_barrier()
      pltpu.sync_copy(acc.at[pl.ds(r0, RPT)], deg_out.at[c, pl.ds(r0, RPT)])

  return pl.kernel(body, mesh=_MESH, out_type=out_type, scratch_types=scratch)


_agg_plain = _make_agg(False)
_agg_deg = _make_agg(True)


def _layer_body(relu, s0_ref, s1_ref, d0_ref, d1_ref, w_ref, b_ref, o_ref):
  d0 = d0_ref[:, 0:1]
  d1 = d1_ref[:, 0:1]
  a0 = s0_ref[...] / jnp.maximum(d0, 1.0)
  a1 = s1_ref[...] / jnp.maximum(d1, 1.0)
  h = jnp.dot(a0, w_ref[0], preferred_element_type=jnp.float32)
  h = h + jnp.dot(a1, w_ref[1], preferred_element_type=jnp.float32)
  h = h + b_ref[0, 0:1, :] * (d0 > 0) + b_ref[0, 1:2, :] * (d1 > 0)
  if relu:
    h = jnp.where(h >= 0, h, 0.01 * h)
  o_ref[...] = h


def _layer_tc(s0, s1, d0, d1, w_pack, b_pack, relu):
  blk = 1024
  grid = (N_PAD // blk,)
  return pl.pallas_call(
      functools.partial(_layer_body, relu),
      grid=grid,
      in_specs=[
          pl.BlockSpec((blk, D), lambda i: (i, 0)),
          pl.BlockSpec((blk, D), lambda i: (i, 0)),
          pl.BlockSpec((blk, D), lambda i: (i, 0)),
          pl.BlockSpec((blk, D), lambda i: (i, 0)),
          pl.BlockSpec((2, D, D), lambda i: (0, 0, 0)),
          pl.BlockSpec((1, 8, D), lambda i: (0, 0, 0)),
      ],
      out_specs=pl.BlockSpec((blk, D), lambda i: (i, 0)),
      out_shape=jax.ShapeDtypeStruct((N_PAD, D), jnp.float32),
  )(s0, s1, d0, d1, w_pack, b_pack)


def _pack_idx(ei):
  src = ei[0].astype(jnp.int32)
  dst = ei[1].astype(jnp.int32)
  pad = NS * EPW - E
  src_p = jnp.concatenate([src, jnp.zeros((pad,), jnp.int32)])
  dst_p = jnp.concatenate([dst, jnp.full((pad,), N, jnp.int32)])
  return (src_p.reshape(NS, CH // HCH, HCH, CHUNK),
          dst_p.reshape(NS, CH // HCH, HCH, CHUNK))


def kernel(embed, edge_index_rel0, edge_index_rel1,
           W1_rel0, b1_rel0, W1_rel1, b1_rel1,
           W2_rel0, b2_rel0, W2_rel1, b2_rel1):
  s0s, d0s = _pack_idx(edge_index_rel0)
  s1s, d1s = _pack_idx(edge_index_rel1)
  src_all = jnp.stack([s0s, s1s])
  dst_all = jnp.stack([d0s, d1s])

  const128 = jnp.zeros((CHUNK, D), jnp.float32).at[:, 0].set(1.0)
  sums1, deg = _agg_deg(embed, src_all, dst_all, const128)

  w1 = jnp.stack([W1_rel0, W1_rel1])
  b1 = jnp.stack([b1_rel0, b1_rel1]).reshape(2, D)
  b1 = jnp.concatenate([b1, jnp.zeros((6, D), jnp.float32)]).reshape(1, 8, D)
  h = _layer_tc(sums1[0], sums1[1], deg[0], deg[1], w1, b1, True)

  sums2 = _agg_plain(h, src_all, dst_all)

  w2 = jnp.stack([W2_rel0, W2_rel1])
  b2 = jnp.stack([b2_rel0, b2_rel1]).reshape(2, D)
  b2 = jnp.concatenate([b2, jnp.zeros((6, D), jnp.float32)]).reshape(1, 8, D)
  out = _layer_tc(sums2[0], sums2[1], deg[0], deg[1], w2, b2, False)
  return out[:N]


# speedup vs baseline: 1.0585x; 1.0130x over previous
import functools

import jax
import jax.numpy as jnp
from jax import lax
from jax.experimental import pallas as pl
from jax.experimental.pallas import tpu as pltpu
from jax.experimental.pallas import tpu_sc as plsc

N = 10000
E = 160000
D = 128

NC = 2
NS = 16
CHUNK = 128
CH = 80
EPW = CH * CHUNK
N_PAD = 10240
RPT = N_PAD // NS


HCH = CH // 2

_MESH = plsc.VectorSubcoreMesh(core_axis_name="c", subcore_axis_name="s")


def _make_agg(with_deg):
  if with_deg:
    out_type = [jax.ShapeDtypeStruct((NC, N_PAD, D), jnp.float32),
                jax.ShapeDtypeStruct((NC, N_PAD, D), jnp.float32)]
  else:
    out_type = jax.ShapeDtypeStruct((NC, N_PAD, D), jnp.float32)
  scratch = [
      pltpu.VMEM_SHARED((N_PAD, D), jnp.float32),
      pltpu.VMEM((HCH, CHUNK), jnp.int32),
      pltpu.VMEM((HCH, CHUNK), jnp.int32),
  ] + [pltpu.VMEM((CHUNK, D), jnp.float32)] * 2 + [
      pltpu.SemaphoreType.DMA] * 2

  def body(*refs):
    if with_deg:
      (x_hbm, src_hbm, dst_hbm, const_hbm, s_out, deg_out,
       acc, src_v, dst_v, r0b, r1b, g0, g1) = refs
    else:
      (x_hbm, src_hbm, dst_hbm, s_out,
       acc, src_v, dst_v, r0b, r1b, g0, g1) = refs
    rows = [r0b, r1b]
    gsem = [g0, g1]
    rows0 = r0b
    c = lax.axis_index("c")
    s = lax.axis_index("s")
    r0 = s * RPT
    zv = jnp.zeros((16,), jnp.float32)

    def zrow(i, carry):
      for k in range(D // 16):
        rows0[i, pl.ds(k * 16, 16)] = zv
      return carry
    def zero_acc():
      lax.fori_loop(0, CHUNK, zrow, 0)
      def zacc(t, carry):
        pltpu.sync_copy(rows0, acc.at[pl.ds(r0 + t * CHUNK, CHUNK)])
        return carry
      lax.fori_loop(0, RPT // CHUNK, zacc, 0)
    zero_acc()
    plsc.subcore_barrier()

    def pair(p, carry):
      j0 = 2 * p
      pltpu.async_copy(x_hbm.at[src_v.at[j0 + 1]], rows[1], gsem[1])
      pltpu.make_async_copy(x_hbm.at[src_v.at[j0]], rows[0], gsem[0]).wait()
      pltpu.sync_copy(rows[0], acc.at[dst_v.at[j0]], add=True)

      @pl.when(j0 + 2 < HCH)
      def _():
        pltpu.async_copy(x_hbm.at[src_v.at[j0 + 2]], rows[0], gsem[0])

      pltpu.make_async_copy(x_hbm.at[src_v.at[j0 + 1]], rows[1],
                            gsem[1]).wait()
      pltpu.sync_copy(rows[1], acc.at[dst_v.at[j0 + 1]], add=True)
      return carry

    for hh in range(2):
      pltpu.sync_copy(src_hbm.at[c, s, hh], src_v)
      pltpu.sync_copy(dst_hbm.at[c, s, hh], dst_v)
      pltpu.async_copy(x_hbm.at[src_v.at[0]], rows[0], gsem[0])
      lax.fori_loop(0, HCH // 2, pair, 0)
    plsc.subcore_barrier()

    pltpu.sync_copy(acc.at[pl.ds(r0, RPT)], s_out.at[c, pl.ds(r0, RPT)])

    if with_deg:
      pltpu.sync_copy(const_hbm, rows[1])
      zero_acc()
      plsc.subcore_barrier()

      def dchunk(j, carry):
        pltpu.sync_copy(rows[1], acc.at[dst_v.at[j]], add=True)
        return carry

      for hh in range(2):
        pltpu.sync_copy(dst_hbm.at[c, s, hh], dst_v)
        lax.fori_loop(0, HCH, dchunk, 0)
      plsc.subcore_barrier()
      pltpu.sync_copy(acc.at[pl.ds(r0, RPT)], deg_out.at[c, pl.ds(r0, RPT)])

  return pl.kernel(body, mesh=_MESH, out_type=out_type, scratch_types=scratch)


_agg_plain = _make_agg(False)
_agg_deg = _make_agg(True)


def _layer_body(relu, s0_ref, s1_ref, d0_ref, d1_ref, w_ref, b_ref, o_ref):
  d0 = d0_ref[:, 0:1]
  d1 = d1_ref[:, 0:1]
  a0 = s0_ref[...] / jnp.maximum(d0, 1.0)
  a1 = s1_ref[...] / jnp.maximum(d1, 1.0)
  h = jnp.dot(a0, w_ref[0], preferred_element_type=jnp.float32)
  h = h + jnp.dot(a1, w_ref[1], preferred_element_type=jnp.float32)
  h = h + b_ref[0, 0:1, :] * (d0 > 0) + b_ref[0, 1:2, :] * (d1 > 0)
  if relu:
    h = jnp.where(h >= 0, h, 0.01 * h)
  o_ref[...] = h


def _layer_tc(s0, s1, d0, d1, w_pack, b_pack, relu):
  blk = 1024
  grid = (N_PAD // blk,)
  return pl.pallas_call(
      functools.partial(_layer_body, relu),
      grid=grid,
      in_specs=[
          pl.BlockSpec((blk, D), lambda i: (i, 0)),
          pl.BlockSpec((blk, D), lambda i: (i, 0)),
          pl.BlockSpec((blk, D), lambda i: (i, 0)),
          pl.BlockSpec((blk, D), lambda i: (i, 0)),
          pl.BlockSpec((2, D, D), lambda i: (0, 0, 0)),
          pl.BlockSpec((1, 8, D), lambda i: (0, 0, 0)),
      ],
      out_specs=pl.BlockSpec((blk, D), lambda i: (i, 0)),
      out_shape=jax.ShapeDtypeStruct((N_PAD, D), jnp.float32),
  )(s0, s1, d0, d1, w_pack, b_pack)


def _pack_idx(ei):
  src = ei[0].astype(jnp.int32)
  dst = ei[1].astype(jnp.int32)
  pad = NS * EPW - E
  src_p = jnp.concatenate([src, jnp.zeros((pad,), jnp.int32)])
  dst_p = jnp.concatenate([dst, jnp.full((pad,), N, jnp.int32)])
  return (src_p.reshape(NS, CH // HCH, HCH, CHUNK),
          dst_p.reshape(NS, CH // HCH, HCH, CHUNK))


def kernel(embed, edge_index_rel0, edge_index_rel1,
           W1_rel0, b1_rel0, W1_rel1, b1_rel1,
           W2_rel0, b2_rel0, W2_rel1, b2_rel1):
  s0s, d0s = _pack_idx(edge_index_rel0)
  s1s, d1s = _pack_idx(edge_index_rel1)
  src_all = jnp.stack([s0s, s1s])
  dst_all = jnp.stack([d0s, d1s])

  const128 = jnp.zeros((CHUNK, D), jnp.float32).at[:, 0].set(1.0)
  sums1, deg = _agg_deg(embed, src_all, dst_all, const128)

  w1 = jnp.stack([W1_rel0, W1_rel1])
  b1 = jnp.stack([b1_rel0, b1_rel1]).reshape(2, D)
  b1 = jnp.concatenate([b1, jnp.zeros((6, D), jnp.float32)]).reshape(1, 8, D)
  h = _layer_tc(sums1[0], sums1[1], deg[0], deg[1], w1, b1, True)

  sums2 = _agg_plain(h, src_all, dst_all)

  w2 = jnp.stack([W2_rel0, W2_rel1])
  b2 = jnp.stack([b2_rel0, b2_rel1]).reshape(2, D)
  b2 = jnp.concatenate([b2, jnp.zeros((6, D), jnp.float32)]).reshape(1, 8, D)
  out = _layer_tc(sums2[0], sums2[1], deg[0], deg[1], w2, b2, False)
  return out[:N]


# speedup vs baseline: 1.0620x; 1.0033x over previous
import functools

import jax
import jax.numpy as jnp
from jax import lax
from jax.experimental import pallas as pl
from jax.experimental.pallas import tpu as pltpu
from jax.experimental.pallas import tpu_sc as plsc

N = 10000
E = 160000
D = 128

NC = 2
NS = 16
CHUNK = 128
CH = 80
EPW = CH * CHUNK
N_PAD = 10240
RPT = N_PAD // NS


HCH = CH // 2

_MESH = plsc.VectorSubcoreMesh(core_axis_name="c", subcore_axis_name="s")


def _make_agg(with_deg):
  if with_deg:
    out_type = [jax.ShapeDtypeStruct((NC, N_PAD, D), jnp.float32),
                jax.ShapeDtypeStruct((NC, N_PAD, D), jnp.float32)]
  else:
    out_type = jax.ShapeDtypeStruct((NC, N_PAD, D), jnp.float32)
  scratch = [
      pltpu.VMEM_SHARED((N_PAD, D), jnp.float32),
      pltpu.VMEM((HCH, CHUNK), jnp.int32),
      pltpu.VMEM((HCH, CHUNK), jnp.int32),
  ] + [pltpu.VMEM((CHUNK, D), jnp.float32)] * 2 + [
      pltpu.SemaphoreType.DMA] * 2

  def body(*refs):
    if with_deg:
      (x_hbm, src_hbm, dst_hbm, const_hbm, s_out, deg_out,
       acc, src_v, dst_v, r0b, r1b, g0, g1) = refs
    else:
      (x_hbm, src_hbm, dst_hbm, s_out,
       acc, src_v, dst_v, r0b, r1b, g0, g1) = refs
    rows = [r0b, r1b]
    gsem = [g0, g1]
    rows0 = r0b
    c = lax.axis_index("c")
    s = lax.axis_index("s")
    r0 = s * RPT
    zv = jnp.zeros((16,), jnp.float32)

    def zrow(i, carry):
      for k in range(D // 16):
        rows0[i, pl.ds(k * 16, 16)] = zv
      return carry
    def zero_acc():
      lax.fori_loop(0, CHUNK, zrow, 0)
      def zacc(t, carry):
        pltpu.sync_copy(rows0, acc.at[pl.ds(r0 + t * CHUNK, CHUNK)])
        return carry
      lax.fori_loop(0, RPT // CHUNK, zacc, 0)
    zero_acc()
    plsc.subcore_barrier()

    def pair(p, carry):
      j0 = 2 * p
      pltpu.async_copy(x_hbm.at[src_v.at[j0 + 1]], rows[1], gsem[1])
      pltpu.make_async_copy(x_hbm.at[src_v.at[j0]], rows[0], gsem[0]).wait()
      pltpu.sync_copy(rows[0], acc.at[dst_v.at[j0]], add=True)

      @pl.when(j0 + 2 < HCH)
      def _():
        pltpu.async_copy(x_hbm.at[src_v.at[j0 + 2]], rows[0], gsem[0])

      pltpu.make_async_copy(x_hbm.at[src_v.at[j0 + 1]], rows[1],
                            gsem[1]).wait()
      pltpu.sync_copy(rows[1], acc.at[dst_v.at[j0 + 1]], add=True)
      return carry

    for hh in range(2):
      pltpu.sync_copy(src_hbm.at[c, s, hh], src_v)
      pltpu.sync_copy(dst_hbm.at[c, s, hh], dst_v)
      pltpu.async_copy(x_hbm.at[src_v.at[0]], rows[0], gsem[0])
      lax.fori_loop(0, HCH // 2, pair, 0)
    plsc.subcore_barrier()

    pltpu.sync_copy(acc.at[pl.ds(r0, RPT)], s_out.at[c, pl.ds(r0, RPT)])

    if with_deg:
      pltpu.sync_copy(const_hbm, rows[1])
      zero_acc()
      plsc.subcore_barrier()

      def dchunk(j, carry):
        pltpu.sync_copy(rows[1], acc.at[dst_v.at[j]], add=True)
        return carry

      for hh in range(2):
        pltpu.sync_copy(dst_hbm.at[c, s, hh], dst_v)
        lax.fori_loop(0, HCH, dchunk, 0)
      plsc.subcore_barrier()
      pltpu.sync_copy(acc.at[pl.ds(r0, RPT)], deg_out.at[c, pl.ds(r0, RPT)])

  return pl.kernel(body, mesh=_MESH, out_type=out_type, scratch_types=scratch)


_agg_plain = _make_agg(False)

def _make_deg():
  out_type = jax.ShapeDtypeStruct((NC, N_PAD, D), jnp.float32)
  scratch = [
      pltpu.VMEM_SHARED((N_PAD, D), jnp.float32),
      pltpu.VMEM((HCH, CHUNK), jnp.int32),
      pltpu.VMEM((CHUNK, D), jnp.float32),
  ]

  def body(dst_hbm, const_hbm, deg_out, acc, dst_v, ones_v):
    c = lax.axis_index("c")
    s = lax.axis_index("s")
    r0 = s * RPT

    def zrow(i, carry):
      for k in range(D // 16):
        ones_v[i, pl.ds(k * 16, 16)] = jnp.zeros((16,), jnp.float32)
      return carry
    lax.fori_loop(0, CHUNK, zrow, 0)

    def zacc(t, carry):
      pltpu.sync_copy(ones_v, acc.at[pl.ds(r0 + t * CHUNK, CHUNK)])
      return carry
    lax.fori_loop(0, RPT // CHUNK, zacc, 0)

    pltpu.sync_copy(const_hbm, ones_v)
    plsc.subcore_barrier()

    def dchunk(j, carry):
      pltpu.sync_copy(ones_v, acc.at[dst_v.at[j]], add=True)
      return carry

    for hh in range(2):
      pltpu.sync_copy(dst_hbm.at[c, s, hh], dst_v)
      lax.fori_loop(0, HCH, dchunk, 0)
    plsc.subcore_barrier()
    pltpu.sync_copy(acc.at[pl.ds(r0, RPT)], deg_out.at[c, pl.ds(r0, RPT)])

  return pl.kernel(body, mesh=_MESH, out_type=out_type, scratch_types=scratch)


_deg_kernel = _make_deg()


def _layer_body(relu, s0_ref, s1_ref, d0_ref, d1_ref, w_ref, b_ref, o_ref):
  d0 = d0_ref[:, 0:1]
  d1 = d1_ref[:, 0:1]
  a0 = s0_ref[...] / jnp.maximum(d0, 1.0)
  a1 = s1_ref[...] / jnp.maximum(d1, 1.0)
  h = jnp.dot(a0, w_ref[0], preferred_element_type=jnp.float32)
  h = h + jnp.dot(a1, w_ref[1], preferred_element_type=jnp.float32)
  h = h + b_ref[0, 0:1, :] * (d0 > 0) + b_ref[0, 1:2, :] * (d1 > 0)
  if relu:
    h = jnp.where(h >= 0, h, 0.01 * h)
  o_ref[...] = h


def _layer_tc(s0, s1, d0, d1, w_pack, b_pack, relu):
  blk = 1024
  grid = (N_PAD // blk,)
  return pl.pallas_call(
      functools.partial(_layer_body, relu),
      grid=grid,
      in_specs=[
          pl.BlockSpec((blk, D), lambda i: (i, 0)),
          pl.BlockSpec((blk, D), lambda i: (i, 0)),
          pl.BlockSpec((blk, D), lambda i: (i, 0)),
          pl.BlockSpec((blk, D), lambda i: (i, 0)),
          pl.BlockSpec((2, D, D), lambda i: (0, 0, 0)),
          pl.BlockSpec((1, 8, D), lambda i: (0, 0, 0)),
      ],
      out_specs=pl.BlockSpec((blk, D), lambda i: (i, 0)),
      out_shape=jax.ShapeDtypeStruct((N_PAD, D), jnp.float32),
  )(s0, s1, d0, d1, w_pack, b_pack)


def _mm_body(s0_ref, s1_ref, w_ref, t0_ref, t1_ref):
  t0_ref[...] = jnp.dot(s0_ref[...], w_ref[0],
                        preferred_element_type=jnp.float32)
  t1_ref[...] = jnp.dot(s1_ref[...], w_ref[1],
                        preferred_element_type=jnp.float32)


def _matmul_tc(s0, s1, w_pack):
  blk = 1024
  return pl.pallas_call(
      _mm_body,
      grid=(N_PAD // blk,),
      in_specs=[
          pl.BlockSpec((blk, D), lambda i: (i, 0)),
          pl.BlockSpec((blk, D), lambda i: (i, 0)),
          pl.BlockSpec((2, D, D), lambda i: (0, 0, 0)),
      ],
      out_specs=[pl.BlockSpec((blk, D), lambda i: (i, 0)),
                 pl.BlockSpec((blk, D), lambda i: (i, 0))],
      out_shape=[jax.ShapeDtypeStruct((N_PAD, D), jnp.float32),
                 jax.ShapeDtypeStruct((N_PAD, D), jnp.float32)],
  )(s0, s1, w_pack)


def _finish_body(t0_ref, t1_ref, d0_ref, d1_ref, b_ref, o_ref):
  d0 = d0_ref[:, 0:1]
  d1 = d1_ref[:, 0:1]
  h = t0_ref[...] / jnp.maximum(d0, 1.0) + t1_ref[...] / jnp.maximum(d1, 1.0)
  h = h + b_ref[0, 0:1, :] * (d0 > 0) + b_ref[0, 1:2, :] * (d1 > 0)
  o_ref[...] = jnp.where(h >= 0, h, 0.01 * h)


def _finish_tc(t0, t1, d0, d1, b_pack):
  blk = 1024
  return pl.pallas_call(
      _finish_body,
      grid=(N_PAD // blk,),
      in_specs=[
          pl.BlockSpec((blk, D), lambda i: (i, 0)),
          pl.BlockSpec((blk, D), lambda i: (i, 0)),
          pl.BlockSpec((blk, D), lambda i: (i, 0)),
          pl.BlockSpec((blk, D), lambda i: (i, 0)),
          pl.BlockSpec((1, 8, D), lambda i: (0, 0, 0)),
      ],
      out_specs=pl.BlockSpec((blk, D), lambda i: (i, 0)),
      out_shape=jax.ShapeDtypeStruct((N_PAD, D), jnp.float32),
  )(t0, t1, d0, d1, b_pack)


def _pack_idx(ei):
  src = ei[0].astype(jnp.int32)
  dst = ei[1].astype(jnp.int32)
  pad = NS * EPW - E
  src_p = jnp.concatenate([src, jnp.zeros((pad,), jnp.int32)])
  dst_p = jnp.concatenate([dst, jnp.full((pad,), N, jnp.int32)])
  return (src_p.reshape(NS, CH // HCH, HCH, CHUNK),
          dst_p.reshape(NS, CH // HCH, HCH, CHUNK))


def kernel(embed, edge_index_rel0, edge_index_rel1,
           W1_rel0, b1_rel0, W1_rel1, b1_rel1,
           W2_rel0, b2_rel0, W2_rel1, b2_rel1):
  s0s, d0s = _pack_idx(edge_index_rel0)
  s1s, d1s = _pack_idx(edge_index_rel1)
  src_all = jnp.stack([s0s, s1s])
  dst_all = jnp.stack([d0s, d1s])

  const128 = jnp.zeros((CHUNK, D), jnp.float32).at[:, 0].set(1.0)
  sums1 = _agg_plain(embed, src_all, dst_all)
  deg = _deg_kernel(dst_all, const128)

  w1 = jnp.stack([W1_rel0, W1_rel1])
  b1 = jnp.stack([b1_rel0, b1_rel1]).reshape(2, D)
  b1 = jnp.concatenate([b1, jnp.zeros((6, D), jnp.float32)]).reshape(1, 8, D)
  t0, t1 = _matmul_tc(sums1[0], sums1[1], w1)
  h = _finish_tc(t0, t1, deg[0], deg[1], b1)

  sums2 = _agg_plain(h, src_all, dst_all)

  w2 = jnp.stack([W2_rel0, W2_rel1])
  b2 = jnp.stack([b2_rel0, b2_rel1]).reshape(2, D)
  b2 = jnp.concatenate([b2, jnp.zeros((6, D), jnp.float32)]).reshape(1, 8, D)
  out = _layer_tc(sums2[0], sums2[1], deg[0], deg[1], w2, b2, False)
  return out[:N]
